# speedup vs baseline: 1.9669x; 1.9669x over previous
import functools

import jax
import jax.numpy as jnp
from jax.experimental import pallas as pl
from jax.experimental.pallas import tpu as pltpu


def _lstm_body(H, x_ref, h_ref, c_ref, wx_ref, wh_ref, b_ref, wo_ref,
               bo_ref, out_ref, hnew_ref):
    g = jnp.dot(x_ref[...], wx_ref[...], preferred_element_type=jnp.float32)
    g = g + jnp.dot(h_ref[...], wh_ref[...], preferred_element_type=jnp.float32)
    g = g + b_ref[...]
    i = jax.nn.sigmoid(g[:, :H])
    o = jax.nn.sigmoid(g[:, H:2 * H])
    f = jax.nn.sigmoid(g[:, 2 * H:3 * H])
    z = jnp.tanh(g[:, 3 * H:])
    c_new = i * z + f * c_ref[...]
    hn = o * jnp.tanh(c_new)
    hnew_ref[...] = hn
    out_ref[...] = (
        jnp.dot(hn, wo_ref[...], preferred_element_type=jnp.float32)
        + bo_ref[...])


@jax.jit
def kernel(inp, h, c, Wx, bx, Wh, Wout, bout):
    B, I = inp.shape
    H = h.shape[1]
    O = Wout.shape[0]

    Wxt = jnp.transpose(Wx, (2, 0, 1)).reshape(I, 4 * H)
    Wht = jnp.transpose(Wh, (2, 0, 1)).reshape(H, 4 * H)
    b = bx.reshape(1, 4 * H)
    WoT = Wout.T
    bo = bout.reshape(1, O)

    BM = 256
    grid = (B // BM,)

    out, h_new = pl.pallas_call(
        functools.partial(_lstm_body, H),
        grid=grid,
        in_specs=[
            pl.BlockSpec((BM, I), lambda b_: (b_, 0)),
            pl.BlockSpec((BM, H), lambda b_: (b_, 0)),
            pl.BlockSpec((BM, H), lambda b_: (b_, 0)),
            pl.BlockSpec((I, 4 * H), lambda b_: (0, 0)),
            pl.BlockSpec((H, 4 * H), lambda b_: (0, 0)),
            pl.BlockSpec((1, 4 * H), lambda b_: (0, 0)),
            pl.BlockSpec((H, O), lambda b_: (0, 0)),
            pl.BlockSpec((1, O), lambda b_: (0, 0)),
        ],
        out_specs=[
            pl.BlockSpec((BM, O), lambda b_: (b_, 0)),
            pl.BlockSpec((BM, H), lambda b_: (b_, 0)),
        ],
        out_shape=[
            jax.ShapeDtypeStruct((B, O), jnp.float32),
            jax.ShapeDtypeStruct((B, H), jnp.float32),
        ],
        compiler_params=pltpu.CompilerParams(
            dimension_semantics=("parallel",),
            vmem_limit_bytes=64 * 1024 * 1024,
        ),
    )(inp, h, c, Wxt, Wht, b, WoT, bo)
    return (out, h_new)


# speedup vs baseline: 2.0479x; 1.0412x over previous
import functools

import jax
import jax.numpy as jnp
from jax.experimental import pallas as pl
from jax.experimental.pallas import tpu as pltpu


def _lstm_body(H, x_ref, h_ref, c_ref, wx_ref, wh_ref, b_ref, wo_ref,
               bo_ref, out_ref, hnew_ref):
    xb = x_ref[...].astype(jnp.bfloat16)
    hb = h_ref[...].astype(jnp.bfloat16)
    g = jnp.dot(xb, wx_ref[...], preferred_element_type=jnp.float32)
    g = g + jnp.dot(hb, wh_ref[...], preferred_element_type=jnp.float32)
    g = g + b_ref[...]
    i = jax.nn.sigmoid(g[:, :H])
    o = jax.nn.sigmoid(g[:, H:2 * H])
    f = jax.nn.sigmoid(g[:, 2 * H:3 * H])
    z = jnp.tanh(g[:, 3 * H:])
    c_new = i * z + f * c_ref[...]
    hn = o * jnp.tanh(c_new)
    hnew_ref[...] = hn
    out_ref[...] = (
        jnp.dot(hn.astype(jnp.bfloat16), wo_ref[...],
                preferred_element_type=jnp.float32)
        + bo_ref[...])


@jax.jit
def kernel(inp, h, c, Wx, bx, Wh, Wout, bout):
    B, I = inp.shape
    H = h.shape[1]
    O = Wout.shape[0]

    Wxt = jnp.transpose(Wx, (2, 0, 1)).reshape(I, 4 * H).astype(jnp.bfloat16)
    Wht = jnp.transpose(Wh, (2, 0, 1)).reshape(H, 4 * H).astype(jnp.bfloat16)
    b = bx.reshape(1, 4 * H)
    WoT = Wout.T.astype(jnp.bfloat16)
    bo = bout.reshape(1, O)

    BM = 256
    grid = (B // BM,)

    out, h_new = pl.pallas_call(
        functools.partial(_lstm_body, H),
        grid=grid,
        in_specs=[
            pl.BlockSpec((BM, I), lambda b_: (b_, 0)),
            pl.BlockSpec((BM, H), lambda b_: (b_, 0)),
            pl.BlockSpec((BM, H), lambda b_: (b_, 0)),
            pl.BlockSpec((I, 4 * H), lambda b_: (0, 0)),
            pl.BlockSpec((H, 4 * H), lambda b_: (0, 0)),
            pl.BlockSpec((1, 4 * H), lambda b_: (0, 0)),
            pl.BlockSpec((H, O), lambda b_: (0, 0)),
            pl.BlockSpec((1, O), lambda b_: (0, 0)),
        ],
        out_specs=[
            pl.BlockSpec((BM, O), lambda b_: (b_, 0)),
            pl.BlockSpec((BM, H), lambda b_: (b_, 0)),
        ],
        out_shape=[
            jax.ShapeDtypeStruct((B, O), jnp.float32),
            jax.ShapeDtypeStruct((B, H), jnp.float32),
        ],
        compiler_params=pltpu.CompilerParams(
            dimension_semantics=("parallel",),
            vmem_limit_bytes=64 * 1024 * 1024,
        ),
    )(inp, h, c, Wxt, Wht, b, WoT, bo)
    return (out, h_new)


# speedup vs baseline: 2.0596x; 1.0057x over previous
import functools

import jax
import jax.numpy as jnp
from jax.experimental import pallas as pl
from jax.experimental.pallas import tpu as pltpu


def _dot_t(a, w):
    return jax.lax.dot_general(a, w, (((1,), (1,)), ((), ())),
                               preferred_element_type=jnp.float32)


def _lstm_body(H, x_ref, h_ref, c_ref, wx_ref, wh_ref, b_ref, wo_ref,
               bo_ref, out_ref, hnew_ref):
    xb = x_ref[...].astype(jnp.bfloat16)
    hb = h_ref[...].astype(jnp.bfloat16)
    g = _dot_t(xb, wx_ref[...]) + _dot_t(hb, wh_ref[...]) + b_ref[...]
    i = jax.nn.sigmoid(g[:, :H])
    o = jax.nn.sigmoid(g[:, H:2 * H])
    f = jax.nn.sigmoid(g[:, 2 * H:3 * H])
    z = jnp.tanh(g[:, 3 * H:])
    c_new = i * z + f * c_ref[...]
    hn = o * jnp.tanh(c_new)
    hnew_ref[...] = hn
    out_ref[...] = _dot_t(hn.astype(jnp.bfloat16), wo_ref[...]) + bo_ref[...]


@jax.jit
def kernel(inp, h, c, Wx, bx, Wh, Wout, bout):
    B, I = inp.shape
    H = h.shape[1]
    O = Wout.shape[0]

    WxR = Wx.reshape(4 * H, I).astype(jnp.bfloat16)
    WhR = Wh.reshape(4 * H, H).astype(jnp.bfloat16)
    b = bx.reshape(1, 4 * H)
    WoB = Wout.astype(jnp.bfloat16)
    bo = bout.reshape(1, O)

    BM = 256
    grid = (B // BM,)

    out, h_new = pl.pallas_call(
        functools.partial(_lstm_body, H),
        grid=grid,
        in_specs=[
            pl.BlockSpec((BM, I), lambda b_: (b_, 0)),
            pl.BlockSpec((BM, H), lambda b_: (b_, 0)),
            pl.BlockSpec((BM, H), lambda b_: (b_, 0)),
            pl.BlockSpec((4 * H, I), lambda b_: (0, 0)),
            pl.BlockSpec((4 * H, H), lambda b_: (0, 0)),
            pl.BlockSpec((1, 4 * H), lambda b_: (0, 0)),
            pl.BlockSpec((O, H), lambda b_: (0, 0)),
            pl.BlockSpec((1, O), lambda b_: (0, 0)),
        ],
        out_specs=[
            pl.BlockSpec((BM, O), lambda b_: (b_, 0)),
            pl.BlockSpec((BM, H), lambda b_: (b_, 0)),
        ],
        out_shape=[
            jax.ShapeDtypeStruct((B, O), jnp.float32),
            jax.ShapeDtypeStruct((B, H), jnp.float32),
        ],
        compiler_params=pltpu.CompilerParams(
            dimension_semantics=("parallel",),
            vmem_limit_bytes=64 * 1024 * 1024,
        ),
    )(inp, h, c, WxR, WhR, b, WoB, bo)
    return (out, h_new)
